# Initial kernel scaffold; baseline (speedup 1.0000x reference)
import jax
import jax.numpy as jnp
from jax.experimental import pallas as pl


def kernel(x, batch_ids, y, W1, b1, W2, b2):
    raise NotImplementedError("write your pallas kernel here")


# speedup vs baseline: 4.7543x; 4.7543x over previous
import jax
import jax.numpy as jnp
from jax.experimental import pallas as pl
from jax.experimental.pallas import tpu as pltpu

N_NODES = 100000
D_IN = 128
NUM_GRAPHS = 512
D_OUT = 32

_R = 800
_NBLK = N_NODES // _R
_AUG = D_IN + 8


def _seg_mlp_kernel(ids_ref, x_ref, w1_ref, b1_ref, w2_ref, b2_ref,
                    out_ref, acc_ref):
    b = pl.program_id(0)

    @pl.when(b == 0)
    def _init():
        acc_ref[...] = jnp.zeros_like(acc_ref)

    ids_row = ids_ref[0, 0, :]
    seg = jax.lax.broadcasted_iota(jnp.int32, (NUM_GRAPHS, _R), 0)
    oh = (seg == ids_row[None, :]).astype(jnp.float32)
    ones_cols = jnp.ones((_R, _AUG - D_IN), dtype=jnp.float32)
    x_aug = jnp.concatenate([x_ref[...], ones_cols], axis=1)
    acc_ref[...] += jax.lax.dot(oh, x_aug,
                                preferred_element_type=jnp.float32)

    @pl.when(b == _NBLK - 1)
    def _finalize():
        sums = acc_ref[:, :D_IN]
        counts = acc_ref[:, D_IN:D_IN + 1]
        emb = sums / jnp.maximum(counts, 1.0)
        h = jnp.maximum(
            jax.lax.dot(emb, w1_ref[...],
                        preferred_element_type=jnp.float32) + b1_ref[...],
            0.0)
        out_ref[...] = (jax.lax.dot(h, w2_ref[...],
                                    preferred_element_type=jnp.float32)
                        + b2_ref[...])


def kernel(x, batch_ids, y, W1, b1, W2, b2):
    ids = batch_ids.astype(jnp.int32).reshape(_NBLK, 1, _R)
    pred = pl.pallas_call(
        _seg_mlp_kernel,
        grid=(_NBLK,),
        in_specs=[
            pl.BlockSpec((1, 1, _R), lambda b: (b, 0, 0)),
            pl.BlockSpec((_R, D_IN), lambda b: (b, 0)),
            pl.BlockSpec((D_IN, D_IN), lambda b: (0, 0)),
            pl.BlockSpec((1, D_IN), lambda b: (0, 0)),
            pl.BlockSpec((D_IN, D_OUT), lambda b: (0, 0)),
            pl.BlockSpec((1, D_OUT), lambda b: (0, 0)),
        ],
        out_specs=pl.BlockSpec((NUM_GRAPHS, D_OUT), lambda b: (0, 0)),
        out_shape=jax.ShapeDtypeStruct((NUM_GRAPHS, D_OUT), jnp.float32),
        scratch_shapes=[pltpu.VMEM((NUM_GRAPHS, _AUG), jnp.float32)],
        compiler_params=pltpu.CompilerParams(
            dimension_semantics=("arbitrary",)),
    )(ids, x, W1, b1.reshape(1, D_IN), W2, b2.reshape(1, D_OUT))
    return (pred, y)
